# Initial kernel scaffold; baseline (speedup 1.0000x reference)
import jax
import jax.numpy as jnp
from jax.experimental import pallas as pl


def kernel(Smiles_node_feature, Smiles_graphs, Smiles_edge_feature, Smiles_graph_ids, elu1_smiles_node_feature, elu1_smiles_graphs, elu1_smiles_edge_feature, elu2_smiles_node_feature, elu2_smiles_graphs, elu2_smiles_edge_feature, params):
    raise NotImplementedError("write your pallas kernel here")


# speedup vs baseline: 16.7031x; 16.7031x over previous
import jax
import jax.numpy as jnp
from jax import lax
from jax.experimental import pallas as pl
from jax.experimental.pallas import tpu as pltpu
from jax.experimental.pallas import tpu_sc as plsc

N = 50000
E = 800000
D = 128
HID = 64
HH = 32
G = 256
LAYERS = 4

K_SCALE = 2.5

NC, NS = 2, 16
NW = NC * NS

CROWS1 = 10
CHUNK1 = CROWS1 * 128
NCHUNK1 = E // CHUNK1

CROWS2 = 5
CHUNK2 = CROWS2 * 128
NCHUNK2 = E // CHUNK2

EROWS = E // 128

_f32 = jnp.float32
_i32 = jnp.int32


def _leaky(x):
    return jnp.where(x >= 0, x, _f32(0.2) * x)


def _over_slices(s, width, total, fn):
    nfull = total // width
    tail = total - nfull * width
    nslices = nfull + (1 if tail else 0)
    for k in range((nslices + NS - 1) // NS):
        z = s + NS * k

        @pl.when(z < nfull)
        def _():
            fn(z * width, width)

        if tail and nfull - NS * k < NS:
            @pl.when(z == nfull)
            def _():
                fn(nfull * width, tail)


def _sc1_body(src_hbm, dst_hbm, as_hbm, ad_hbm, e_out, p_out,
              as_t, ad_t, src2, dst2, eb, tb, zb, p_sh, sem):
    c = lax.axis_index("c")
    s = lax.axis_index("s")
    wid = c * NS + s

    @pl.loop(0, CHUNK1, step=16)
    def _(i):
        zb[pl.ds(i, 16)] = jnp.zeros((16,), _f32)

    _over_slices(s, CHUNK1, N, lambda off, sz: pltpu.sync_copy(
        zb.at[pl.ds(0, sz)], p_sh.at[pl.ds(off, sz)]))

    pltpu.sync_copy(as_hbm, as_t)
    pltpu.sync_copy(ad_hbm, ad_t)
    plsc.subcore_barrier()

    nk = (NCHUNK1 - wid + (NW - 1)) // NW

    @pl.loop(0, nk)
    def _(k):
        z = wid + NW * k
        cp1 = pltpu.async_copy(src_hbm.at[pl.ds(z * CROWS1, CROWS1)], src2, sem)
        cp2 = pltpu.async_copy(dst_hbm.at[pl.ds(z * CROWS1, CROWS1)], dst2, sem)
        cp1.wait()
        cp2.wait()

        @pl.loop(0, CROWS1)
        def _(o):
            @pl.loop(0, 128, step=16)
            def _(g):
                sv = src2[o, pl.ds(g, 16)]
                dv = dst2[o, pl.ds(g, 16)]
                av = plsc.load_gather(as_t, [sv])
                bv = plsc.load_gather(ad_t, [dv])
                ev = _leaky(av + bv)
                eb[o, pl.ds(g, 16)] = ev
                tb[o, pl.ds(g, 16)] = jnp.exp(ev * _f32(1.0 / K_SCALE))

        pltpu.sync_copy(eb, e_out.at[pl.ds(z * CROWS1, CROWS1)])
        for o in range(CROWS1):
            pltpu.sync_copy(tb.at[o], p_sh.at[dst2.at[o]], add=True)

    plsc.subcore_barrier()

    @pl.when(c == 0)
    def _():
        _over_slices(s, CHUNK1, N, lambda off, sz: pltpu.sync_copy(
            p_sh.at[pl.ds(off, sz)], p_out.at[0].at[pl.ds(off, sz)]))

    @pl.when(c == 1)
    def _():
        _over_slices(s, CHUNK1, N, lambda off, sz: pltpu.sync_copy(
            p_sh.at[pl.ds(off, sz)], p_out.at[1].at[pl.ds(off, sz)]))


_SC_CP = pltpu.CompilerParams(use_tc_tiling_on_sc=False,
                              needs_layout_passes=False)

_sc1 = pl.kernel(
    _sc1_body,
    compiler_params=_SC_CP,
    out_type=[jax.ShapeDtypeStruct((EROWS, 128), _f32),
              jax.ShapeDtypeStruct((NC, N), _f32)],
    mesh=plsc.VectorSubcoreMesh(core_axis_name="c", subcore_axis_name="s",
                                num_cores=NC, num_subcores=NS),
    scratch_types=[pltpu.VMEM((N,), _f32),
                   pltpu.VMEM((N,), _f32),
                   pltpu.VMEM((CROWS1, 128), _i32),
                   pltpu.VMEM((CROWS1, 128), _i32),
                   pltpu.VMEM((CROWS1, 128), _f32),
                   pltpu.VMEM((CROWS1, 128), _f32),
                   pltpu.VMEM((CHUNK1,), _f32),
                   pltpu.VMEM_SHARED((N,), _f32),
                   pltpu.SemaphoreType.DMA],
)


def _sc2_body(src_hbm, dst_hbm, e_hbm, c_hbm, whl_hbm, whh_hbm,
              ml_out, mh_out, den_out,
              src2, dst2, eb, cbuf, eeb, rows, msg_sh, den_sh, sem):
    c = lax.axis_index("c")
    s = lax.axis_index("s")

    @pl.loop(0, CHUNK2)
    def _(i):
        rows[i, pl.ds(0, 16)] = jnp.zeros((16,), _f32)
        rows[i, pl.ds(16, 16)] = jnp.zeros((16,), _f32)

    @pl.loop(0, CHUNK2, step=16)
    def _(i):
        eeb[pl.ds(i, 16)] = jnp.zeros((16,), _f32)

    def _zero(off, sz):
        pltpu.sync_copy(rows.at[pl.ds(0, sz)], msg_sh.at[pl.ds(off, sz)])
        pltpu.sync_copy(eeb.at[pl.ds(0, sz)], den_sh.at[pl.ds(off, sz)])

    _over_slices(s, CHUNK2, N, _zero)
    plsc.subcore_barrier()

    nk = (NCHUNK2 - s + (NS - 1)) // NS

    def _chunks(wh_ref, do_den):
        @pl.loop(0, nk)
        def _(k):
            z = s + NS * k
            cp1 = pltpu.async_copy(src_hbm.at[pl.ds(z * CROWS2, CROWS2)],
                                   src2, sem)
            cp2 = pltpu.async_copy(dst_hbm.at[pl.ds(z * CROWS2, CROWS2)],
                                   dst2, sem)
            cp3 = pltpu.async_copy(e_hbm.at[pl.ds(z * CROWS2, CROWS2)],
                                   eb, sem)
            cp1.wait()
            cp2.wait()
            cp3.wait()

            for o in range(CROWS2):
                pltpu.sync_copy(c_hbm.at[dst2.at[o]],
                                cbuf.at[pl.ds(o * 128, 128)])
            for o in range(CROWS2):
                pltpu.sync_copy(wh_ref.at[src2.at[o]],
                                rows.at[pl.ds(o * 128, 128)])

            @pl.loop(0, CROWS2)
            def _(o):
                @pl.loop(0, 128, step=16)
                def _(g):
                    ev = eb[o, pl.ds(g, 16)]
                    cg = cbuf[pl.ds(o * 128 + g, 16)]
                    eeb[pl.ds(o * 128 + g, 16)] = jnp.exp(ev - cg)

            @pl.loop(0, CHUNK2, step=4)
            def _(j):
                for jj in range(4):
                    f = plsc.load_gather(eeb, [jnp.full((16,), j + jj, _i32)])
                    rows[j + jj, pl.ds(0, 16)] = rows[j + jj, pl.ds(0, 16)] * f
                    rows[j + jj, pl.ds(16, 16)] = (
                        rows[j + jj, pl.ds(16, 16)] * f)

            for o in range(CROWS2):
                pltpu.sync_copy(rows.at[pl.ds(o * 128, 128)],
                                msg_sh.at[dst2.at[o]], add=True)
            if do_den:
                for o in range(CROWS2):
                    pltpu.sync_copy(eeb.at[pl.ds(o * 128, 128)],
                                    den_sh.at[dst2.at[o]], add=True)

    @pl.when(c == 0)
    def _():
        _chunks(whl_hbm, True)

    @pl.when(c == 1)
    def _():
        _chunks(whh_hbm, False)

    plsc.subcore_barrier()

    def _wb_lo(off, sz):
        pltpu.sync_copy(msg_sh.at[pl.ds(off, sz)], ml_out.at[pl.ds(off, sz)])
        pltpu.sync_copy(den_sh.at[pl.ds(off, sz)], den_out.at[pl.ds(off, sz)])

    def _wb_hi(off, sz):
        pltpu.sync_copy(msg_sh.at[pl.ds(off, sz)], mh_out.at[pl.ds(off, sz)])

    @pl.when(c == 0)
    def _():
        _over_slices(s, CHUNK2, N, _wb_lo)

    @pl.when(c == 1)
    def _():
        _over_slices(s, CHUNK2, N, _wb_hi)


_sc2 = pl.kernel(
    _sc2_body,
    compiler_params=_SC_CP,
    out_type=[jax.ShapeDtypeStruct((N, HH), _f32),
              jax.ShapeDtypeStruct((N, HH), _f32),
              jax.ShapeDtypeStruct((N,), _f32)],
    mesh=plsc.VectorSubcoreMesh(core_axis_name="c", subcore_axis_name="s",
                                num_cores=NC, num_subcores=NS),
    scratch_types=[pltpu.VMEM((CROWS2, 128), _i32),
                   pltpu.VMEM((CROWS2, 128), _i32),
                   pltpu.VMEM((CROWS2, 128), _f32),
                   pltpu.VMEM((CHUNK2,), _f32),
                   pltpu.VMEM((CHUNK2,), _f32),
                   pltpu.VMEM((CHUNK2, HH), _f32),
                   pltpu.VMEM_SHARED((N, HH), _f32),
                   pltpu.VMEM_SHARED((N,), _f32),
                   pltpu.SemaphoreType.DMA],
)


_BLK = 2000


def _tca0_body(x_ref, we_ref, be_ref, w_ref, asrc_ref, adst_ref,
               whl_ref, whh_ref, as_ref, ad_ref):
    h = jnp.dot(x_ref[...], we_ref[...], preferred_element_type=_f32)
    h = h + be_ref[...]
    wh = jnp.dot(h, w_ref[...], preferred_element_type=_f32)
    whl_ref[...] = wh[:, :HH]
    whh_ref[...] = wh[:, HH:]
    as_ref[...] = jnp.sum(wh * asrc_ref[...], axis=1, keepdims=True)
    ad_ref[...] = jnp.sum(wh * adst_ref[...], axis=1, keepdims=True)


def _tcam_body(ml_ref, mh_ref, den_ref, w_ref, asrc_ref, adst_ref,
               whl_ref, whh_ref, as_ref, ad_ref):
    msg = jnp.concatenate([ml_ref[...], mh_ref[...]], axis=1)
    h = jax.nn.relu(msg / (den_ref[...] + _f32(1e-9)))
    wh = jnp.dot(h, w_ref[...], preferred_element_type=_f32)
    whl_ref[...] = wh[:, :HH]
    whh_ref[...] = wh[:, HH:]
    as_ref[...] = jnp.sum(wh * asrc_ref[...], axis=1, keepdims=True)
    ad_ref[...] = jnp.sum(wh * adst_ref[...], axis=1, keepdims=True)


def _tcb_body(p_ref, c_ref):
    c_ref[...] = _f32(K_SCALE) * jnp.log(p_ref[0:1, :] + p_ref[1:2, :])


def _pool_body(ml_ref, mh_ref, den_ref, gid_ref,
               w1_ref, b1_ref, w2_ref, b2_ref, w3_ref, b3_ref,
               out_ref, acc_ref):
    i = pl.program_id(0)

    @pl.when(i == 0)
    def _():
        acc_ref[...] = jnp.zeros_like(acc_ref)

    msg = jnp.concatenate([ml_ref[...], mh_ref[...]], axis=1)
    h = jax.nn.relu(msg / (den_ref[...] + _f32(1e-9)))
    onehot = (gid_ref[...] ==
              lax.broadcasted_iota(_i32, (_BLK, G), 1)).astype(_f32)
    acc_ref[...] += lax.dot_general(onehot, h, (((0,), (0,)), ((), ())),
                                    preferred_element_type=_f32)

    @pl.when(i == pl.num_programs(0) - 1)
    def _():
        z = jax.nn.relu(jnp.dot(acc_ref[...], w1_ref[...],
                                preferred_element_type=_f32) + b1_ref[...])
        z = jax.nn.relu(jnp.dot(z, w2_ref[...],
                                preferred_element_type=_f32) + b2_ref[...])
        out_ref[...] = jnp.dot(z, w3_ref[...],
                               preferred_element_type=_f32) + b3_ref[...]


def _row_spec(width):
    return pl.BlockSpec((_BLK, width), lambda i: (i, 0))


def _full_spec(shape):
    return pl.BlockSpec(shape, lambda i: tuple(0 for _ in shape))


def _tca0(x, we, be, w, asrc, adst):
    return pl.pallas_call(
        _tca0_body,
        grid=(N // _BLK,),
        in_specs=[_row_spec(D), _full_spec((D, HID)), _full_spec((1, HID)),
                  _full_spec((HID, HID)), _full_spec((1, HID)),
                  _full_spec((1, HID))],
        out_specs=[_row_spec(HH), _row_spec(HH), _row_spec(1), _row_spec(1)],
        out_shape=[jax.ShapeDtypeStruct((N, HH), _f32),
                   jax.ShapeDtypeStruct((N, HH), _f32),
                   jax.ShapeDtypeStruct((N, 1), _f32),
                   jax.ShapeDtypeStruct((N, 1), _f32)],
    )(x, we, be, w, asrc, adst)


def _tcam(ml, mh, den, w, asrc, adst):
    return pl.pallas_call(
        _tcam_body,
        grid=(N // _BLK,),
        in_specs=[_row_spec(HH), _row_spec(HH), _row_spec(1),
                  _full_spec((HID, HID)), _full_spec((1, HID)),
                  _full_spec((1, HID))],
        out_specs=[_row_spec(HH), _row_spec(HH), _row_spec(1), _row_spec(1)],
        out_shape=[jax.ShapeDtypeStruct((N, HH), _f32),
                   jax.ShapeDtypeStruct((N, HH), _f32),
                   jax.ShapeDtypeStruct((N, 1), _f32),
                   jax.ShapeDtypeStruct((N, 1), _f32)],
    )(ml, mh, den, w, asrc, adst)


def _tcb(p):
    return pl.pallas_call(
        _tcb_body,
        out_shape=jax.ShapeDtypeStruct((1, N), _f32),
    )(p)


def _pool(ml, mh, den, gid2, w1, b1, w2, b2, w3, b3):
    return pl.pallas_call(
        _pool_body,
        grid=(N // _BLK,),
        in_specs=[_row_spec(HH), _row_spec(HH), _row_spec(1), _row_spec(1),
                  _full_spec((HID, 32)), _full_spec((1, 32)),
                  _full_spec((32, 16)), _full_spec((1, 16)),
                  _full_spec((16, 1)), _full_spec((1, 1))],
        out_specs=pl.BlockSpec((G, 1), lambda i: (0, 0)),
        out_shape=jax.ShapeDtypeStruct((G, 1), _f32),
        scratch_shapes=[pltpu.VMEM((G, HID), _f32)],
    )(ml, mh, den, gid2, w1, b1, w2, b2, w3, b3)


@jax.jit
def _run(x, edge_index, gid, params):
    src2d = edge_index[0].reshape(EROWS, 128)
    dst2d = edge_index[1].reshape(EROWS, 128)
    gid2 = gid.reshape(N, 1)

    ml = mh = den = None
    for i in range(LAYERS):
        w = params["gat_W%d" % i]
        asrc = params["gat_asrc%d" % i].reshape(1, HID)
        adst = params["gat_adst%d" % i].reshape(1, HID)
        if i == 0:
            whl, whh, a_s, a_d = _tca0(
                x, params["W_embed"], params["b_embed"].reshape(1, HID),
                w, asrc, adst)
        else:
            whl, whh, a_s, a_d = _tcam(ml, mh, den, w, asrc, adst)
        e, p = _sc1(src2d, dst2d, a_s.reshape(N), a_d.reshape(N))
        cvec = _tcb(p).reshape(N)
        ml, mh, den = _sc2(src2d, dst2d, e, cvec, whl, whh)
        den = den.reshape(N, 1)

    return _pool(ml, mh, den, gid2,
                 params["W1"], params["b1"].reshape(1, 32),
                 params["W2"], params["b2"].reshape(1, 16),
                 params["W3"], params["b3"].reshape(1, 1))


def kernel(Smiles_node_feature, Smiles_graphs, Smiles_edge_feature,
           Smiles_graph_ids, elu1_smiles_node_feature, elu1_smiles_graphs,
           elu1_smiles_edge_feature, elu2_smiles_node_feature,
           elu2_smiles_graphs, elu2_smiles_edge_feature, params):
    return _run(Smiles_node_feature, Smiles_graphs, Smiles_graph_ids, params)


# speedup vs baseline: 27.6095x; 1.6530x over previous
import jax
import jax.numpy as jnp
from jax import lax
from jax.experimental import pallas as pl
from jax.experimental.pallas import tpu as pltpu
from jax.experimental.pallas import tpu_sc as plsc

N = 50000
E = 800000
D = 128
HID = 64
HH = 32
G = 256
LAYERS = 4

K_SCALE = 2.5

NC, NS = 2, 16
NW = NC * NS

CROWS1 = 10
CHUNK1 = CROWS1 * 128
NCHUNK1 = E // CHUNK1

CROWS2 = 5
CHUNK2 = CROWS2 * 128
NCHUNK2 = E // CHUNK2

EROWS = E // 128

_f32 = jnp.float32
_i32 = jnp.int32


def _leaky(x):
    return jnp.where(x >= 0, x, _f32(0.2) * x)


def _over_slices(s, width, total, fn):
    nfull = total // width
    tail = total - nfull * width
    nslices = nfull + (1 if tail else 0)
    for k in range((nslices + NS - 1) // NS):
        z = s + NS * k

        @pl.when(z < nfull)
        def _():
            fn(z * width, width)

        if tail and nfull - NS * k < NS:
            @pl.when(z == nfull)
            def _():
                fn(nfull * width, tail)


def _sc1_body(src_hbm, dst_hbm, as_hbm, ad_hbm, e_out, p_out,
              as_t, ad_t, src2, dst2, eb, tb, zb, p_sh, sem, sem_sc):
    c = lax.axis_index("c")
    s = lax.axis_index("s")
    wid = c * NS + s

    @pl.loop(0, CHUNK1, step=16)
    def _(i):
        zb[pl.ds(i, 16)] = jnp.zeros((16,), _f32)

    _over_slices(s, CHUNK1, N, lambda off, sz: pltpu.sync_copy(
        zb.at[pl.ds(0, sz)], p_sh.at[pl.ds(off, sz)]))

    pltpu.sync_copy(as_hbm, as_t)
    pltpu.sync_copy(ad_hbm, ad_t)
    plsc.subcore_barrier()

    nk = (NCHUNK1 - wid + (NW - 1)) // NW

    @pl.loop(0, nk)
    def _(k):
        z = wid + NW * k
        cp1 = pltpu.async_copy(src_hbm.at[pl.ds(z * CROWS1, CROWS1)], src2, sem)
        cp2 = pltpu.async_copy(dst_hbm.at[pl.ds(z * CROWS1, CROWS1)], dst2, sem)
        cp1.wait()
        cp2.wait()

        @pl.loop(0, CROWS1)
        def _(o):
            @pl.loop(0, 128, step=16)
            def _(g):
                sv = src2[o, pl.ds(g, 16)]
                dv = dst2[o, pl.ds(g, 16)]
                av = plsc.load_gather(as_t, [sv])
                bv = plsc.load_gather(ad_t, [dv])
                ev = _leaky(av + bv)
                eb[o, pl.ds(g, 16)] = ev
                tb[o, pl.ds(g, 16)] = jnp.exp(ev * _f32(1.0 / K_SCALE))

        cpe = pltpu.async_copy(eb, e_out.at[pl.ds(z * CROWS1, CROWS1)], sem)
        scs = [pltpu.async_copy(tb.at[o], p_sh.at[dst2.at[o]], sem_sc,
                                add=True)
               for o in range(CROWS1)]
        cpe.wait()
        for cp in scs:
            cp.wait()

    plsc.subcore_barrier()

    @pl.when(c == 0)
    def _():
        _over_slices(s, CHUNK1, N, lambda off, sz: pltpu.sync_copy(
            p_sh.at[pl.ds(off, sz)], p_out.at[0].at[pl.ds(off, sz)]))

    @pl.when(c == 1)
    def _():
        _over_slices(s, CHUNK1, N, lambda off, sz: pltpu.sync_copy(
            p_sh.at[pl.ds(off, sz)], p_out.at[1].at[pl.ds(off, sz)]))


_SC_CP = pltpu.CompilerParams(use_tc_tiling_on_sc=False,
                              needs_layout_passes=False)

_sc1 = pl.kernel(
    _sc1_body,
    compiler_params=_SC_CP,
    out_type=[jax.ShapeDtypeStruct((EROWS, 128), _f32),
              jax.ShapeDtypeStruct((NC, N), _f32)],
    mesh=plsc.VectorSubcoreMesh(core_axis_name="c", subcore_axis_name="s",
                                num_cores=NC, num_subcores=NS),
    scratch_types=[pltpu.VMEM((N,), _f32),
                   pltpu.VMEM((N,), _f32),
                   pltpu.VMEM((CROWS1, 128), _i32),
                   pltpu.VMEM((CROWS1, 128), _i32),
                   pltpu.VMEM((CROWS1, 128), _f32),
                   pltpu.VMEM((CROWS1, 128), _f32),
                   pltpu.VMEM((CHUNK1,), _f32),
                   pltpu.VMEM_SHARED((N,), _f32),
                   pltpu.SemaphoreType.DMA,
                   pltpu.SemaphoreType.DMA],
)


def _sc2_body(src_hbm, dst_hbm, e_hbm, c_hbm, whl_hbm, whh_hbm,
              ml_out, mh_out, den_out,
              src2, dst2, eb, cbuf, eeb, rows, msg_sh, den_sh,
              sem, sem_cg, sem_rg, sem_sc):
    c = lax.axis_index("c")
    s = lax.axis_index("s")

    @pl.loop(0, CHUNK2)
    def _(i):
        rows[i, pl.ds(0, 16)] = jnp.zeros((16,), _f32)
        rows[i, pl.ds(16, 16)] = jnp.zeros((16,), _f32)

    @pl.loop(0, CHUNK2, step=16)
    def _(i):
        eeb[pl.ds(i, 16)] = jnp.zeros((16,), _f32)

    def _zero(off, sz):
        pltpu.sync_copy(rows.at[pl.ds(0, sz)], msg_sh.at[pl.ds(off, sz)])
        pltpu.sync_copy(eeb.at[pl.ds(0, sz)], den_sh.at[pl.ds(off, sz)])

    _over_slices(s, CHUNK2, N, _zero)
    plsc.subcore_barrier()

    nk = (NCHUNK2 - s + (NS - 1)) // NS

    def _chunks(wh_ref, do_den):
        @pl.loop(0, nk)
        def _(k):
            z = s + NS * k
            cp1 = pltpu.async_copy(src_hbm.at[pl.ds(z * CROWS2, CROWS2)],
                                   src2, sem)
            cp2 = pltpu.async_copy(dst_hbm.at[pl.ds(z * CROWS2, CROWS2)],
                                   dst2, sem)
            cp3 = pltpu.async_copy(e_hbm.at[pl.ds(z * CROWS2, CROWS2)],
                                   eb, sem)
            cp1.wait()
            cp2.wait()
            cp3.wait()

            cgs = [pltpu.async_copy(c_hbm.at[dst2.at[o]],
                                    cbuf.at[pl.ds(o * 128, 128)], sem_cg)
                   for o in range(CROWS2)]
            rgs = [pltpu.async_copy(wh_ref.at[src2.at[o]],
                                    rows.at[pl.ds(o * 128, 128)], sem_rg)
                   for o in range(CROWS2)]
            for cp in cgs:
                cp.wait()

            @pl.loop(0, CROWS2)
            def _(o):
                @pl.loop(0, 128, step=16)
                def _(g):
                    ev = eb[o, pl.ds(g, 16)]
                    cg = cbuf[pl.ds(o * 128 + g, 16)]
                    eeb[pl.ds(o * 128 + g, 16)] = jnp.exp(ev - cg)

            for cp in rgs:
                cp.wait()

            @pl.loop(0, CHUNK2, step=4)
            def _(j):
                for jj in range(4):
                    f = plsc.load_gather(eeb, [jnp.full((16,), j + jj, _i32)])
                    rows[j + jj, pl.ds(0, 16)] = rows[j + jj, pl.ds(0, 16)] * f
                    rows[j + jj, pl.ds(16, 16)] = (
                        rows[j + jj, pl.ds(16, 16)] * f)

            scs = [pltpu.async_copy(rows.at[pl.ds(o * 128, 128)],
                                    msg_sh.at[dst2.at[o]], sem_sc, add=True)
                   for o in range(CROWS2)]
            if do_den:
                scs += [pltpu.async_copy(eeb.at[pl.ds(o * 128, 128)],
                                         den_sh.at[dst2.at[o]], sem_cg,
                                         add=True)
                        for o in range(CROWS2)]
            for cp in scs:
                cp.wait()

    @pl.when(c == 0)
    def _():
        _chunks(whl_hbm, True)

    @pl.when(c == 1)
    def _():
        _chunks(whh_hbm, False)

    plsc.subcore_barrier()

    def _wb_lo(off, sz):
        pltpu.sync_copy(msg_sh.at[pl.ds(off, sz)], ml_out.at[pl.ds(off, sz)])
        pltpu.sync_copy(den_sh.at[pl.ds(off, sz)], den_out.at[pl.ds(off, sz)])

    def _wb_hi(off, sz):
        pltpu.sync_copy(msg_sh.at[pl.ds(off, sz)], mh_out.at[pl.ds(off, sz)])

    @pl.when(c == 0)
    def _():
        _over_slices(s, CHUNK2, N, _wb_lo)

    @pl.when(c == 1)
    def _():
        _over_slices(s, CHUNK2, N, _wb_hi)


_sc2 = pl.kernel(
    _sc2_body,
    compiler_params=_SC_CP,
    out_type=[jax.ShapeDtypeStruct((N, HH), _f32),
              jax.ShapeDtypeStruct((N, HH), _f32),
              jax.ShapeDtypeStruct((N,), _f32)],
    mesh=plsc.VectorSubcoreMesh(core_axis_name="c", subcore_axis_name="s",
                                num_cores=NC, num_subcores=NS),
    scratch_types=[pltpu.VMEM((CROWS2, 128), _i32),
                   pltpu.VMEM((CROWS2, 128), _i32),
                   pltpu.VMEM((CROWS2, 128), _f32),
                   pltpu.VMEM((CHUNK2,), _f32),
                   pltpu.VMEM((CHUNK2,), _f32),
                   pltpu.VMEM((CHUNK2, HH), _f32),
                   pltpu.VMEM_SHARED((N, HH), _f32),
                   pltpu.VMEM_SHARED((N,), _f32),
                   pltpu.SemaphoreType.DMA,
                   pltpu.SemaphoreType.DMA,
                   pltpu.SemaphoreType.DMA,
                   pltpu.SemaphoreType.DMA],
)


_BLK = 2000


def _tca0_body(x_ref, we_ref, be_ref, w_ref, asrc_ref, adst_ref,
               whl_ref, whh_ref, as_ref, ad_ref):
    h = jnp.dot(x_ref[...], we_ref[...], preferred_element_type=_f32)
    h = h + be_ref[...]
    wh = jnp.dot(h, w_ref[...], preferred_element_type=_f32)
    whl_ref[...] = wh[:, :HH]
    whh_ref[...] = wh[:, HH:]
    as_ref[...] = jnp.sum(wh * asrc_ref[...], axis=1, keepdims=True)
    ad_ref[...] = jnp.sum(wh * adst_ref[...], axis=1, keepdims=True)


def _tcam_body(ml_ref, mh_ref, den_ref, w_ref, asrc_ref, adst_ref,
               whl_ref, whh_ref, as_ref, ad_ref):
    msg = jnp.concatenate([ml_ref[...], mh_ref[...]], axis=1)
    h = jax.nn.relu(msg / (den_ref[...] + _f32(1e-9)))
    wh = jnp.dot(h, w_ref[...], preferred_element_type=_f32)
    whl_ref[...] = wh[:, :HH]
    whh_ref[...] = wh[:, HH:]
    as_ref[...] = jnp.sum(wh * asrc_ref[...], axis=1, keepdims=True)
    ad_ref[...] = jnp.sum(wh * adst_ref[...], axis=1, keepdims=True)


def _tcb_body(p_ref, c_ref):
    c_ref[...] = _f32(K_SCALE) * jnp.log(p_ref[0:1, :] + p_ref[1:2, :])


def _pool_body(ml_ref, mh_ref, den_ref, gid_ref,
               w1_ref, b1_ref, w2_ref, b2_ref, w3_ref, b3_ref,
               out_ref, acc_ref):
    i = pl.program_id(0)

    @pl.when(i == 0)
    def _():
        acc_ref[...] = jnp.zeros_like(acc_ref)

    msg = jnp.concatenate([ml_ref[...], mh_ref[...]], axis=1)
    h = jax.nn.relu(msg / (den_ref[...] + _f32(1e-9)))
    onehot = (gid_ref[...] ==
              lax.broadcasted_iota(_i32, (_BLK, G), 1)).astype(_f32)
    acc_ref[...] += lax.dot_general(onehot, h, (((0,), (0,)), ((), ())),
                                    preferred_element_type=_f32)

    @pl.when(i == pl.num_programs(0) - 1)
    def _():
        z = jax.nn.relu(jnp.dot(acc_ref[...], w1_ref[...],
                                preferred_element_type=_f32) + b1_ref[...])
        z = jax.nn.relu(jnp.dot(z, w2_ref[...],
                                preferred_element_type=_f32) + b2_ref[...])
        out_ref[...] = jnp.dot(z, w3_ref[...],
                               preferred_element_type=_f32) + b3_ref[...]


def _row_spec(width):
    return pl.BlockSpec((_BLK, width), lambda i: (i, 0))


def _full_spec(shape):
    return pl.BlockSpec(shape, lambda i: tuple(0 for _ in shape))


def _tca0(x, we, be, w, asrc, adst):
    return pl.pallas_call(
        _tca0_body,
        grid=(N // _BLK,),
        in_specs=[_row_spec(D), _full_spec((D, HID)), _full_spec((1, HID)),
                  _full_spec((HID, HID)), _full_spec((1, HID)),
                  _full_spec((1, HID))],
        out_specs=[_row_spec(HH), _row_spec(HH), _row_spec(1), _row_spec(1)],
        out_shape=[jax.ShapeDtypeStruct((N, HH), _f32),
                   jax.ShapeDtypeStruct((N, HH), _f32),
                   jax.ShapeDtypeStruct((N, 1), _f32),
                   jax.ShapeDtypeStruct((N, 1), _f32)],
    )(x, we, be, w, asrc, adst)


def _tcam(ml, mh, den, w, asrc, adst):
    return pl.pallas_call(
        _tcam_body,
        grid=(N // _BLK,),
        in_specs=[_row_spec(HH), _row_spec(HH), _row_spec(1),
                  _full_spec((HID, HID)), _full_spec((1, HID)),
                  _full_spec((1, HID))],
        out_specs=[_row_spec(HH), _row_spec(HH), _row_spec(1), _row_spec(1)],
        out_shape=[jax.ShapeDtypeStruct((N, HH), _f32),
                   jax.ShapeDtypeStruct((N, HH), _f32),
                   jax.ShapeDtypeStruct((N, 1), _f32),
                   jax.ShapeDtypeStruct((N, 1), _f32)],
    )(ml, mh, den, w, asrc, adst)


def _tcb(p):
    return pl.pallas_call(
        _tcb_body,
        out_shape=jax.ShapeDtypeStruct((1, N), _f32),
    )(p)


def _pool(ml, mh, den, gid2, w1, b1, w2, b2, w3, b3):
    return pl.pallas_call(
        _pool_body,
        grid=(N // _BLK,),
        in_specs=[_row_spec(HH), _row_spec(HH), _row_spec(1), _row_spec(1),
                  _full_spec((HID, 32)), _full_spec((1, 32)),
                  _full_spec((32, 16)), _full_spec((1, 16)),
                  _full_spec((16, 1)), _full_spec((1, 1))],
        out_specs=pl.BlockSpec((G, 1), lambda i: (0, 0)),
        out_shape=jax.ShapeDtypeStruct((G, 1), _f32),
        scratch_shapes=[pltpu.VMEM((G, HID), _f32)],
    )(ml, mh, den, gid2, w1, b1, w2, b2, w3, b3)


@jax.jit
def _run(x, edge_index, gid, params):
    src2d = edge_index[0].reshape(EROWS, 128)
    dst2d = edge_index[1].reshape(EROWS, 128)
    gid2 = gid.reshape(N, 1)

    ml = mh = den = None
    for i in range(LAYERS):
        w = params["gat_W%d" % i]
        asrc = params["gat_asrc%d" % i].reshape(1, HID)
        adst = params["gat_adst%d" % i].reshape(1, HID)
        if i == 0:
            whl, whh, a_s, a_d = _tca0(
                x, params["W_embed"], params["b_embed"].reshape(1, HID),
                w, asrc, adst)
        else:
            whl, whh, a_s, a_d = _tcam(ml, mh, den, w, asrc, adst)
        e, p = _sc1(src2d, dst2d, a_s.reshape(N), a_d.reshape(N))
        cvec = _tcb(p).reshape(N)
        ml, mh, den = _sc2(src2d, dst2d, e, cvec, whl, whh)
        den = den.reshape(N, 1)

    return _pool(ml, mh, den, gid2,
                 params["W1"], params["b1"].reshape(1, 32),
                 params["W2"], params["b2"].reshape(1, 16),
                 params["W3"], params["b3"].reshape(1, 1))


def kernel(Smiles_node_feature, Smiles_graphs, Smiles_edge_feature,
           Smiles_graph_ids, elu1_smiles_node_feature, elu1_smiles_graphs,
           elu1_smiles_edge_feature, elu2_smiles_node_feature,
           elu2_smiles_graphs, elu2_smiles_edge_feature, params):
    return _run(Smiles_node_feature, Smiles_graphs, Smiles_graph_ids, params)


# speedup vs baseline: 29.0798x; 1.0533x over previous
import jax
import jax.numpy as jnp
from jax import lax
from jax.experimental import pallas as pl
from jax.experimental.pallas import tpu as pltpu
from jax.experimental.pallas import tpu_sc as plsc

N = 50000
E = 800000
D = 128
HID = 64
HH = 32
G = 256
LAYERS = 4

NC, NS = 2, 16
NW = NC * NS

CROWS1 = 10
CHUNK1 = CROWS1 * 128
NCHUNK1 = E // CHUNK1

CROWS2 = 5
CHUNK2 = CROWS2 * 128
NCHUNK2 = E // CHUNK2

EROWS = E // 128

_f32 = jnp.float32
_i32 = jnp.int32


def _leaky(x):
    return jnp.where(x >= 0, x, _f32(0.2) * x)


def _over_slices(s, width, total, fn):
    nfull = total // width
    tail = total - nfull * width
    nslices = nfull + (1 if tail else 0)
    for k in range((nslices + NS - 1) // NS):
        z = s + NS * k

        @pl.when(z < nfull)
        def _():
            fn(z * width, width)

        if tail and nfull - NS * k < NS:
            @pl.when(z == nfull)
            def _():
                fn(nfull * width, tail)


def _sc1_body(src_hbm, dst_hbm, as_hbm, ad_hbm, w_out,
              as_t, ad_t, src2, dst2, wb, sem, sem_w):
    c = lax.axis_index("c")
    s = lax.axis_index("s")
    wid = c * NS + s

    pltpu.sync_copy(as_hbm, as_t)
    pltpu.sync_copy(ad_hbm, ad_t)

    nk = (NCHUNK1 - wid + (NW - 1)) // NW

    @pl.loop(0, nk)
    def _(k):
        z = wid + NW * k
        cp1 = pltpu.async_copy(src_hbm.at[pl.ds(z * CROWS1, CROWS1)], src2, sem)
        cp2 = pltpu.async_copy(dst_hbm.at[pl.ds(z * CROWS1, CROWS1)], dst2, sem)
        cp1.wait()
        cp2.wait()

        @pl.loop(0, CROWS1)
        def _(o):
            @pl.loop(0, 128, step=16)
            def _(g):
                sv = src2[o, pl.ds(g, 16)]
                dv = dst2[o, pl.ds(g, 16)]
                av = plsc.load_gather(as_t, [sv])
                bv = plsc.load_gather(ad_t, [dv])
                wb[o, pl.ds(g, 16)] = jnp.exp(_leaky(av + bv))

        pltpu.async_copy(wb, w_out.at[pl.ds(z * CROWS1, CROWS1)], sem_w).wait()


_SC_CP = pltpu.CompilerParams(use_tc_tiling_on_sc=False,
                              needs_layout_passes=False)

_sc1 = pl.kernel(
    _sc1_body,
    compiler_params=_SC_CP,
    out_type=jax.ShapeDtypeStruct((EROWS, 128), _f32),
    mesh=plsc.VectorSubcoreMesh(core_axis_name="c", subcore_axis_name="s",
                                num_cores=NC, num_subcores=NS),
    scratch_types=[pltpu.VMEM((N,), _f32),
                   pltpu.VMEM((N,), _f32),
                   pltpu.VMEM((CROWS1, 128), _i32),
                   pltpu.VMEM((CROWS1, 128), _i32),
                   pltpu.VMEM((CROWS1, 128), _f32),
                   pltpu.SemaphoreType.DMA,
                   pltpu.SemaphoreType.DMA],
)


def _sc2_body(src_hbm, dst_hbm, w_hbm, whl_hbm, whh_hbm,
              sl_out, sh_out, t_out,
              src2, dst2, wb, rows, zb, s_sh, t_sh,
              sem, sem_rg, sem_sc, sem_t):
    c = lax.axis_index("c")
    s = lax.axis_index("s")

    @pl.loop(0, CHUNK2)
    def _(i):
        rows[i, pl.ds(0, 16)] = jnp.zeros((16,), _f32)
        rows[i, pl.ds(16, 16)] = jnp.zeros((16,), _f32)

    @pl.loop(0, CHUNK2, step=16)
    def _(i):
        zb[pl.ds(i, 16)] = jnp.zeros((16,), _f32)

    def _zero(off, sz):
        pltpu.sync_copy(rows.at[pl.ds(0, sz)], s_sh.at[pl.ds(off, sz)])
        pltpu.sync_copy(zb.at[pl.ds(0, sz)], t_sh.at[pl.ds(off, sz)])

    _over_slices(s, CHUNK2, N, _zero)
    plsc.subcore_barrier()

    nk = (NCHUNK2 - s + (NS - 1)) // NS

    def _chunks(wh_ref, do_t):
        @pl.loop(0, nk)
        def _(k):
            z = s + NS * k
            cp1 = pltpu.async_copy(src_hbm.at[pl.ds(z * CROWS2, CROWS2)],
                                   src2, sem)
            cp2 = pltpu.async_copy(dst_hbm.at[pl.ds(z * CROWS2, CROWS2)],
                                   dst2, sem)
            cp3 = pltpu.async_copy(w_hbm.at[pl.ds(z * CROWS2, CROWS2)],
                                   wb, sem)
            cp1.wait()
            cp2.wait()
            cp3.wait()

            rgs = [pltpu.async_copy(wh_ref.at[src2.at[o]],
                                    rows.at[pl.ds(o * 128, 128)], sem_rg)
                   for o in range(CROWS2)]

            scs = []
            if do_t:
                scs += [pltpu.async_copy(wb.at[o], t_sh.at[dst2.at[o]],
                                         sem_t, add=True)
                        for o in range(CROWS2)]
            for cp in rgs:
                cp.wait()

            for o in range(CROWS2):
                @pl.loop(0, 128, step=4)
                def _(jr):
                    for jj in range(4):
                        j = o * 128 + jr + jj
                        f = plsc.load_gather(
                            wb, [jnp.full((16,), o, _i32),
                                 jnp.full((16,), jr + jj, _i32)])
                        rows[j, pl.ds(0, 16)] = rows[j, pl.ds(0, 16)] * f
                        rows[j, pl.ds(16, 16)] = rows[j, pl.ds(16, 16)] * f

            scs += [pltpu.async_copy(rows.at[pl.ds(o * 128, 128)],
                                     s_sh.at[dst2.at[o]], sem_sc, add=True)
                    for o in range(CROWS2)]
            for cp in scs:
                cp.wait()

    @pl.when(c == 0)
    def _():
        _chunks(whl_hbm, True)

    @pl.when(c == 1)
    def _():
        _chunks(whh_hbm, False)

    plsc.subcore_barrier()

    def _wb_lo(off, sz):
        pltpu.sync_copy(s_sh.at[pl.ds(off, sz)], sl_out.at[pl.ds(off, sz)])
        pltpu.sync_copy(t_sh.at[pl.ds(off, sz)], t_out.at[pl.ds(off, sz)])

    def _wb_hi(off, sz):
        pltpu.sync_copy(s_sh.at[pl.ds(off, sz)], sh_out.at[pl.ds(off, sz)])

    @pl.when(c == 0)
    def _():
        _over_slices(s, CHUNK2, N, _wb_lo)

    @pl.when(c == 1)
    def _():
        _over_slices(s, CHUNK2, N, _wb_hi)


_sc2 = pl.kernel(
    _sc2_body,
    compiler_params=_SC_CP,
    out_type=[jax.ShapeDtypeStruct((N, HH), _f32),
              jax.ShapeDtypeStruct((N, HH), _f32),
              jax.ShapeDtypeStruct((N,), _f32)],
    mesh=plsc.VectorSubcoreMesh(core_axis_name="c", subcore_axis_name="s",
                                num_cores=NC, num_subcores=NS),
    scratch_types=[pltpu.VMEM((CROWS2, 128), _i32),
                   pltpu.VMEM((CROWS2, 128), _i32),
                   pltpu.VMEM((CROWS2, 128), _f32),
                   pltpu.VMEM((CHUNK2, HH), _f32),
                   pltpu.VMEM((CHUNK2,), _f32),
                   pltpu.VMEM_SHARED((N, HH), _f32),
                   pltpu.VMEM_SHARED((N,), _f32),
                   pltpu.SemaphoreType.DMA,
                   pltpu.SemaphoreType.DMA,
                   pltpu.SemaphoreType.DMA,
                   pltpu.SemaphoreType.DMA],
)


_BLK = 2000


def _hdiv(sl, sh, t):
    smsg = jnp.concatenate([sl, sh], axis=1)
    d = t + _f32(1e-9) * t
    return jnp.where(t > 0, jax.nn.relu(smsg / d), _f32(0.0))


def _tca0_body(x_ref, we_ref, be_ref, w_ref, asrc_ref, adst_ref,
               whl_ref, whh_ref, as_ref, ad_ref):
    h = jnp.dot(x_ref[...], we_ref[...], preferred_element_type=_f32)
    h = h + be_ref[...]
    wh = jnp.dot(h, w_ref[...], preferred_element_type=_f32)
    whl_ref[...] = wh[:, :HH]
    whh_ref[...] = wh[:, HH:]
    as_ref[...] = jnp.sum(wh * asrc_ref[...], axis=1, keepdims=True)
    ad_ref[...] = jnp.sum(wh * adst_ref[...], axis=1, keepdims=True)


def _tcam_body(sl_ref, sh_ref, t_ref, w_ref, asrc_ref, adst_ref,
               whl_ref, whh_ref, as_ref, ad_ref):
    h = _hdiv(sl_ref[...], sh_ref[...], t_ref[...])
    wh = jnp.dot(h, w_ref[...], preferred_element_type=_f32)
    whl_ref[...] = wh[:, :HH]
    whh_ref[...] = wh[:, HH:]
    as_ref[...] = jnp.sum(wh * asrc_ref[...], axis=1, keepdims=True)
    ad_ref[...] = jnp.sum(wh * adst_ref[...], axis=1, keepdims=True)


def _pool_body(sl_ref, sh_ref, t_ref, gid_ref,
               w1_ref, b1_ref, w2_ref, b2_ref, w3_ref, b3_ref,
               out_ref, acc_ref):
    i = pl.program_id(0)

    @pl.when(i == 0)
    def _():
        acc_ref[...] = jnp.zeros_like(acc_ref)

    h = _hdiv(sl_ref[...], sh_ref[...], t_ref[...])
    onehot = (gid_ref[...] ==
              lax.broadcasted_iota(_i32, (_BLK, G), 1)).astype(_f32)
    acc_ref[...] += lax.dot_general(onehot, h, (((0,), (0,)), ((), ())),
                                    preferred_element_type=_f32)

    @pl.when(i == pl.num_programs(0) - 1)
    def _():
        z = jax.nn.relu(jnp.dot(acc_ref[...], w1_ref[...],
                                preferred_element_type=_f32) + b1_ref[...])
        z = jax.nn.relu(jnp.dot(z, w2_ref[...],
                                preferred_element_type=_f32) + b2_ref[...])
        out_ref[...] = jnp.dot(z, w3_ref[...],
                               preferred_element_type=_f32) + b3_ref[...]


def _row_spec(width):
    return pl.BlockSpec((_BLK, width), lambda i: (i, 0))


def _full_spec(shape):
    return pl.BlockSpec(shape, lambda i: tuple(0 for _ in shape))


def _tca0(x, we, be, w, asrc, adst):
    return pl.pallas_call(
        _tca0_body,
        grid=(N // _BLK,),
        in_specs=[_row_spec(D), _full_spec((D, HID)), _full_spec((1, HID)),
                  _full_spec((HID, HID)), _full_spec((1, HID)),
                  _full_spec((1, HID))],
        out_specs=[_row_spec(HH), _row_spec(HH), _row_spec(1), _row_spec(1)],
        out_shape=[jax.ShapeDtypeStruct((N, HH), _f32),
                   jax.ShapeDtypeStruct((N, HH), _f32),
                   jax.ShapeDtypeStruct((N, 1), _f32),
                   jax.ShapeDtypeStruct((N, 1), _f32)],
    )(x, we, be, w, asrc, adst)


def _tcam(sl, sh, t, w, asrc, adst):
    return pl.pallas_call(
        _tcam_body,
        grid=(N // _BLK,),
        in_specs=[_row_spec(HH), _row_spec(HH), _row_spec(1),
                  _full_spec((HID, HID)), _full_spec((1, HID)),
                  _full_spec((1, HID))],
        out_specs=[_row_spec(HH), _row_spec(HH), _row_spec(1), _row_spec(1)],
        out_shape=[jax.ShapeDtypeStruct((N, HH), _f32),
                   jax.ShapeDtypeStruct((N, HH), _f32),
                   jax.ShapeDtypeStruct((N, 1), _f32),
                   jax.ShapeDtypeStruct((N, 1), _f32)],
    )(sl, sh, t, w, asrc, adst)


def _pool(sl, sh, t, gid2, w1, b1, w2, b2, w3, b3):
    return pl.pallas_call(
        _pool_body,
        grid=(N // _BLK,),
        in_specs=[_row_spec(HH), _row_spec(HH), _row_spec(1), _row_spec(1),
                  _full_spec((HID, 32)), _full_spec((1, 32)),
                  _full_spec((32, 16)), _full_spec((1, 16)),
                  _full_spec((16, 1)), _full_spec((1, 1))],
        out_specs=pl.BlockSpec((G, 1), lambda i: (0, 0)),
        out_shape=jax.ShapeDtypeStruct((G, 1), _f32),
        scratch_shapes=[pltpu.VMEM((G, HID), _f32)],
    )(sl, sh, t, gid2, w1, b1, w2, b2, w3, b3)


@jax.jit
def _run(x, edge_index, gid, params):
    src2d = edge_index[0].reshape(EROWS, 128)
    dst2d = edge_index[1].reshape(EROWS, 128)
    gid2 = gid.reshape(N, 1)

    sl = sh = t = None
    for i in range(LAYERS):
        w = params["gat_W%d" % i]
        asrc = params["gat_asrc%d" % i].reshape(1, HID)
        adst = params["gat_adst%d" % i].reshape(1, HID)
        if i == 0:
            whl, whh, a_s, a_d = _tca0(
                x, params["W_embed"], params["b_embed"].reshape(1, HID),
                w, asrc, adst)
        else:
            whl, whh, a_s, a_d = _tcam(sl, sh, t, w, asrc, adst)
        ew = _sc1(src2d, dst2d, a_s.reshape(N), a_d.reshape(N))
        sl, sh, t = _sc2(src2d, dst2d, ew, whl, whh)
        t = t.reshape(N, 1)

    return _pool(sl, sh, t, gid2,
                 params["W1"], params["b1"].reshape(1, 32),
                 params["W2"], params["b2"].reshape(1, 16),
                 params["W3"], params["b3"].reshape(1, 1))


def kernel(Smiles_node_feature, Smiles_graphs, Smiles_edge_feature,
           Smiles_graph_ids, elu1_smiles_node_feature, elu1_smiles_graphs,
           elu1_smiles_edge_feature, elu2_smiles_node_feature,
           elu2_smiles_graphs, elu2_smiles_edge_feature, params):
    return _run(Smiles_node_feature, Smiles_graphs, Smiles_graph_ids, params)


# speedup vs baseline: 35.2758x; 1.2131x over previous
import jax
import jax.numpy as jnp
from jax import lax
from jax.experimental import pallas as pl
from jax.experimental.pallas import tpu as pltpu
from jax.experimental.pallas import tpu_sc as plsc

N = 50000
E = 800000
D = 128
HID = 64
HH = 32
G = 256
LAYERS = 4

NC, NS = 2, 16
NW = NC * NS

CROWS1 = 10
CHUNK1 = CROWS1 * 128
NCHUNK1 = E // CHUNK1

CROWS2 = 5
CHUNK2 = CROWS2 * 128
NCHUNK2 = E // CHUNK2

EROWS = E // 128

_f32 = jnp.float32
_i32 = jnp.int32


def _leaky(x):
    return jnp.where(x >= 0, x, _f32(0.2) * x)


def _over_slices(s, width, total, fn):
    nfull = total // width
    tail = total - nfull * width
    nslices = nfull + (1 if tail else 0)
    for k in range((nslices + NS - 1) // NS):
        z = s + NS * k

        @pl.when(z < nfull)
        def _():
            fn(z * width, width)

        if tail and nfull - NS * k < NS:
            @pl.when(z == nfull)
            def _():
                fn(nfull * width, tail)


def _sc1_body(src_hbm, dst_hbm, as_hbm, ad_hbm, w_out,
              as_t, ad_t, src2, dst2, wb, sem, sem_w):
    c = lax.axis_index("c")
    s = lax.axis_index("s")
    wid = c * NS + s

    pltpu.sync_copy(as_hbm, as_t)
    pltpu.sync_copy(ad_hbm, ad_t)

    nk = (NCHUNK1 - wid + (NW - 1)) // NW

    @pl.loop(0, nk)
    def _(k):
        z = wid + NW * k
        cp1 = pltpu.async_copy(src_hbm.at[pl.ds(z * CROWS1, CROWS1)], src2, sem)
        cp2 = pltpu.async_copy(dst_hbm.at[pl.ds(z * CROWS1, CROWS1)], dst2, sem)
        cp1.wait()
        cp2.wait()

        @pl.loop(0, CROWS1)
        def _(o):
            @pl.loop(0, 128, step=16)
            def _(g):
                sv = src2[o, pl.ds(g, 16)]
                dv = dst2[o, pl.ds(g, 16)]
                av = plsc.load_gather(as_t, [sv])
                bv = plsc.load_gather(ad_t, [dv])
                wb[o, pl.ds(g, 16)] = jnp.exp(_leaky(av + bv))

        pltpu.async_copy(wb, w_out.at[pl.ds(z * CROWS1, CROWS1)], sem_w).wait()


_SC_CP = pltpu.CompilerParams(use_tc_tiling_on_sc=False,
                              needs_layout_passes=False)

_sc1 = pl.kernel(
    _sc1_body,
    compiler_params=_SC_CP,
    out_type=jax.ShapeDtypeStruct((EROWS, 128), _f32),
    mesh=plsc.VectorSubcoreMesh(core_axis_name="c", subcore_axis_name="s",
                                num_cores=NC, num_subcores=NS),
    scratch_types=[pltpu.VMEM((N,), _f32),
                   pltpu.VMEM((N,), _f32),
                   pltpu.VMEM((CROWS1, 128), _i32),
                   pltpu.VMEM((CROWS1, 128), _i32),
                   pltpu.VMEM((CROWS1, 128), _f32),
                   pltpu.SemaphoreType.DMA,
                   pltpu.SemaphoreType.DMA],
)


def _sc2_body(src_hbm, dst_hbm, w_hbm, whl_hbm, whh_hbm,
              sl_out, sh_out, t_out,
              src2, dst2, wb, rows, zb, s_sh, t_sh,
              sem, sem_rg, sem_sc, sem_t):
    c = lax.axis_index("c")
    s = lax.axis_index("s")

    @pl.loop(0, CHUNK2)
    def _(i):
        rows[i, pl.ds(0, 16)] = jnp.zeros((16,), _f32)
        rows[i, pl.ds(16, 16)] = jnp.zeros((16,), _f32)

    @pl.loop(0, CHUNK2, step=16)
    def _(i):
        zb[pl.ds(i, 16)] = jnp.zeros((16,), _f32)

    def _zero(off, sz):
        pltpu.sync_copy(rows.at[pl.ds(0, sz)], s_sh.at[pl.ds(off, sz)])
        pltpu.sync_copy(zb.at[pl.ds(0, sz)], t_sh.at[pl.ds(off, sz)])

    _over_slices(s, CHUNK2, N, _zero)
    plsc.subcore_barrier()

    nk = (NCHUNK2 - s + (NS - 1)) // NS

    def _chunks(wh_ref, do_t):
        def fire_l(kk, b):
            z = s + NS * kk
            pltpu.async_copy(src_hbm.at[pl.ds(z * CROWS2, CROWS2)],
                             src2.at[b], sem)
            pltpu.async_copy(dst_hbm.at[pl.ds(z * CROWS2, CROWS2)],
                             dst2.at[b], sem)
            pltpu.async_copy(w_hbm.at[pl.ds(z * CROWS2, CROWS2)],
                             wb.at[b], sem)

        def drain_l(b):
            pltpu.make_async_copy(src_hbm.at[pl.ds(0, CROWS2)],
                                  src2.at[b], sem).wait()
            pltpu.make_async_copy(dst_hbm.at[pl.ds(0, CROWS2)],
                                  dst2.at[b], sem).wait()
            pltpu.make_async_copy(w_hbm.at[pl.ds(0, CROWS2)],
                                  wb.at[b], sem).wait()

        @pl.when(nk > 0)
        def _():
            fire_l(0, 0)

        npairs = (nk + 2) // 2

        @pl.loop(0, npairs)
        def _(p):
            for sub in (0, 1):
                kk = 2 * p + sub
                b, bn = sub, 1 - sub

                @pl.when(kk <= nk)
                def _():
                    @pl.when(kk >= 1)
                    def _():
                        if do_t:
                            for o in range(CROWS2):
                                pltpu.make_async_copy(
                                    wb.at[bn].at[o],
                                    t_sh.at[dst2.at[bn].at[o]], sem_t).wait()
                        for o in range(CROWS2):
                            pltpu.make_async_copy(
                                rows.at[pl.ds(o * 128, 128)],
                                s_sh.at[dst2.at[bn].at[o]], sem_sc).wait()

                    @pl.when(kk < nk)
                    def _():
                        drain_l(b)
                        for o in range(CROWS2):
                            pltpu.async_copy(wh_ref.at[src2.at[b].at[o]],
                                             rows.at[pl.ds(o * 128, 128)],
                                             sem_rg)
                        if do_t:
                            for o in range(CROWS2):
                                pltpu.async_copy(wb.at[b].at[o],
                                                 t_sh.at[dst2.at[b].at[o]],
                                                 sem_t, add=True)

                        @pl.when(kk + 1 < nk)
                        def _():
                            fire_l(kk + 1, bn)

                        for o in range(CROWS2):
                            pltpu.make_async_copy(
                                wh_ref.at[src2.at[b].at[o]],
                                rows.at[pl.ds(o * 128, 128)], sem_rg).wait()

                            @pl.loop(0, 128, step=4)
                            def _(jr):
                                for jj in range(4):
                                    j = o * 128 + jr + jj
                                    f = plsc.load_gather(
                                        wb, [jnp.full((16,), b, _i32),
                                             jnp.full((16,), o, _i32),
                                             jnp.full((16,), jr + jj, _i32)])
                                    rows[j, pl.ds(0, 16)] = (
                                        rows[j, pl.ds(0, 16)] * f)
                                    rows[j, pl.ds(16, 16)] = (
                                        rows[j, pl.ds(16, 16)] * f)

                            pltpu.async_copy(rows.at[pl.ds(o * 128, 128)],
                                             s_sh.at[dst2.at[b].at[o]],
                                             sem_sc, add=True)

    @pl.when(c == 0)
    def _():
        _chunks(whl_hbm, True)

    @pl.when(c == 1)
    def _():
        _chunks(whh_hbm, False)

    plsc.subcore_barrier()

    def _wb_lo(off, sz):
        pltpu.sync_copy(s_sh.at[pl.ds(off, sz)], sl_out.at[pl.ds(off, sz)])
        pltpu.sync_copy(t_sh.at[pl.ds(off, sz)], t_out.at[pl.ds(off, sz)])

    def _wb_hi(off, sz):
        pltpu.sync_copy(s_sh.at[pl.ds(off, sz)], sh_out.at[pl.ds(off, sz)])

    @pl.when(c == 0)
    def _():
        _over_slices(s, CHUNK2, N, _wb_lo)

    @pl.when(c == 1)
    def _():
        _over_slices(s, CHUNK2, N, _wb_hi)


_sc2 = pl.kernel(
    _sc2_body,
    compiler_params=_SC_CP,
    out_type=[jax.ShapeDtypeStruct((N, HH), _f32),
              jax.ShapeDtypeStruct((N, HH), _f32),
              jax.ShapeDtypeStruct((N,), _f32)],
    mesh=plsc.VectorSubcoreMesh(core_axis_name="c", subcore_axis_name="s",
                                num_cores=NC, num_subcores=NS),
    scratch_types=[pltpu.VMEM((2, CROWS2, 128), _i32),
                   pltpu.VMEM((2, CROWS2, 128), _i32),
                   pltpu.VMEM((2, CROWS2, 128), _f32),
                   pltpu.VMEM((CHUNK2, HH), _f32),
                   pltpu.VMEM((CHUNK2,), _f32),
                   pltpu.VMEM_SHARED((N, HH), _f32),
                   pltpu.VMEM_SHARED((N,), _f32),
                   pltpu.SemaphoreType.DMA,
                   pltpu.SemaphoreType.DMA,
                   pltpu.SemaphoreType.DMA,
                   pltpu.SemaphoreType.DMA],
)


_BLK = 2000


def _hdiv(sl, sh, t):
    smsg = jnp.concatenate([sl, sh], axis=1)
    d = t + _f32(1e-9) * t
    return jnp.where(t > 0, jax.nn.relu(smsg / d), _f32(0.0))


def _tca0_body(x_ref, we_ref, be_ref, w_ref, asrc_ref, adst_ref,
               whl_ref, whh_ref, as_ref, ad_ref):
    h = jnp.dot(x_ref[...], we_ref[...], preferred_element_type=_f32)
    h = h + be_ref[...]
    wh = jnp.dot(h, w_ref[...], preferred_element_type=_f32)
    whl_ref[...] = wh[:, :HH]
    whh_ref[...] = wh[:, HH:]
    as_ref[...] = jnp.sum(wh * asrc_ref[...], axis=1, keepdims=True)
    ad_ref[...] = jnp.sum(wh * adst_ref[...], axis=1, keepdims=True)


def _tcam_body(sl_ref, sh_ref, t_ref, w_ref, asrc_ref, adst_ref,
               whl_ref, whh_ref, as_ref, ad_ref):
    h = _hdiv(sl_ref[...], sh_ref[...], t_ref[...])
    wh = jnp.dot(h, w_ref[...], preferred_element_type=_f32)
    whl_ref[...] = wh[:, :HH]
    whh_ref[...] = wh[:, HH:]
    as_ref[...] = jnp.sum(wh * asrc_ref[...], axis=1, keepdims=True)
    ad_ref[...] = jnp.sum(wh * adst_ref[...], axis=1, keepdims=True)


def _pool_body(sl_ref, sh_ref, t_ref, gid_ref,
               w1_ref, b1_ref, w2_ref, b2_ref, w3_ref, b3_ref,
               out_ref, acc_ref):
    i = pl.program_id(0)

    @pl.when(i == 0)
    def _():
        acc_ref[...] = jnp.zeros_like(acc_ref)

    h = _hdiv(sl_ref[...], sh_ref[...], t_ref[...])
    onehot = (gid_ref[...] ==
              lax.broadcasted_iota(_i32, (_BLK, G), 1)).astype(_f32)
    acc_ref[...] += lax.dot_general(onehot, h, (((0,), (0,)), ((), ())),
                                    preferred_element_type=_f32)

    @pl.when(i == pl.num_programs(0) - 1)
    def _():
        z = jax.nn.relu(jnp.dot(acc_ref[...], w1_ref[...],
                                preferred_element_type=_f32) + b1_ref[...])
        z = jax.nn.relu(jnp.dot(z, w2_ref[...],
                                preferred_element_type=_f32) + b2_ref[...])
        out_ref[...] = jnp.dot(z, w3_ref[...],
                               preferred_element_type=_f32) + b3_ref[...]


def _row_spec(width):
    return pl.BlockSpec((_BLK, width), lambda i: (i, 0))


def _full_spec(shape):
    return pl.BlockSpec(shape, lambda i: tuple(0 for _ in shape))


def _tca0(x, we, be, w, asrc, adst):
    return pl.pallas_call(
        _tca0_body,
        grid=(N // _BLK,),
        in_specs=[_row_spec(D), _full_spec((D, HID)), _full_spec((1, HID)),
                  _full_spec((HID, HID)), _full_spec((1, HID)),
                  _full_spec((1, HID))],
        out_specs=[_row_spec(HH), _row_spec(HH), _row_spec(1), _row_spec(1)],
        out_shape=[jax.ShapeDtypeStruct((N, HH), _f32),
                   jax.ShapeDtypeStruct((N, HH), _f32),
                   jax.ShapeDtypeStruct((N, 1), _f32),
                   jax.ShapeDtypeStruct((N, 1), _f32)],
    )(x, we, be, w, asrc, adst)


def _tcam(sl, sh, t, w, asrc, adst):
    return pl.pallas_call(
        _tcam_body,
        grid=(N // _BLK,),
        in_specs=[_row_spec(HH), _row_spec(HH), _row_spec(1),
                  _full_spec((HID, HID)), _full_spec((1, HID)),
                  _full_spec((1, HID))],
        out_specs=[_row_spec(HH), _row_spec(HH), _row_spec(1), _row_spec(1)],
        out_shape=[jax.ShapeDtypeStruct((N, HH), _f32),
                   jax.ShapeDtypeStruct((N, HH), _f32),
                   jax.ShapeDtypeStruct((N, 1), _f32),
                   jax.ShapeDtypeStruct((N, 1), _f32)],
    )(sl, sh, t, w, asrc, adst)


def _pool(sl, sh, t, gid2, w1, b1, w2, b2, w3, b3):
    return pl.pallas_call(
        _pool_body,
        grid=(N // _BLK,),
        in_specs=[_row_spec(HH), _row_spec(HH), _row_spec(1), _row_spec(1),
                  _full_spec((HID, 32)), _full_spec((1, 32)),
                  _full_spec((32, 16)), _full_spec((1, 16)),
                  _full_spec((16, 1)), _full_spec((1, 1))],
        out_specs=pl.BlockSpec((G, 1), lambda i: (0, 0)),
        out_shape=jax.ShapeDtypeStruct((G, 1), _f32),
        scratch_shapes=[pltpu.VMEM((G, HID), _f32)],
    )(sl, sh, t, gid2, w1, b1, w2, b2, w3, b3)


@jax.jit
def _run(x, edge_index, gid, params):
    src2d = edge_index[0].reshape(EROWS, 128)
    dst2d = edge_index[1].reshape(EROWS, 128)
    gid2 = gid.reshape(N, 1)

    sl = sh = t = None
    for i in range(LAYERS):
        w = params["gat_W%d" % i]
        asrc = params["gat_asrc%d" % i].reshape(1, HID)
        adst = params["gat_adst%d" % i].reshape(1, HID)
        if i == 0:
            whl, whh, a_s, a_d = _tca0(
                x, params["W_embed"], params["b_embed"].reshape(1, HID),
                w, asrc, adst)
        else:
            whl, whh, a_s, a_d = _tcam(sl, sh, t, w, asrc, adst)
        ew = _sc1(src2d, dst2d, a_s.reshape(N), a_d.reshape(N))
        sl, sh, t = _sc2(src2d, dst2d, ew, whl, whh)
        t = t.reshape(N, 1)

    return _pool(sl, sh, t, gid2,
                 params["W1"], params["b1"].reshape(1, 32),
                 params["W2"], params["b2"].reshape(1, 16),
                 params["W3"], params["b3"].reshape(1, 1))


def kernel(Smiles_node_feature, Smiles_graphs, Smiles_edge_feature,
           Smiles_graph_ids, elu1_smiles_node_feature, elu1_smiles_graphs,
           elu1_smiles_edge_feature, elu2_smiles_node_feature,
           elu2_smiles_graphs, elu2_smiles_edge_feature, params):
    return _run(Smiles_node_feature, Smiles_graphs, Smiles_graph_ids, params)
